# baseline (speedup 1.0000x reference)
import jax
import jax.numpy as jnp
from jax import lax
from jax.experimental import pallas as pl
from jax.experimental.pallas import tpu as pltpu

_B, _T, _H, _W, _D = 8, 8, 14, 14, 768
_HW = _H * _W
_L = _T * _HW
_EPS = 1e-06
_NBUF = 4
_PREF = 2


def _stream_kernel(x_ref, te_ref, ve_ref, he_ref, g_ref, b_ref, o_ref,
                   buf, pos_ref, tbl, in_sem, out_sem, tbl_sem):
    in_h = [None] * _B
    out_h = [None] * _B
    for i in range(_PREF + 1):
        in_h[i] = pltpu.make_async_copy(x_ref.at[i], buf.at[i], in_sem.at[i])
        in_h[i].start()

    cps = [
        pltpu.make_async_copy(te_ref, tbl.at[0, pl.ds(0, _T)], tbl_sem),
        pltpu.make_async_copy(ve_ref, tbl.at[1, pl.ds(0, _H)], tbl_sem),
        pltpu.make_async_copy(he_ref, tbl.at[2, pl.ds(0, _W)], tbl_sem),
        pltpu.make_async_copy(g_ref, tbl.at[3, pl.ds(0, 1)], tbl_sem),
        pltpu.make_async_copy(b_ref, tbl.at[4, pl.ds(0, 1)], tbl_sem),
    ]
    for cp in cps:
        cp.start()
    for cp in cps:
        cp.wait()

    def onehot(idx_fn, n):
        row = lax.broadcasted_iota(jnp.int32, (_L, n), 0)
        col = lax.broadcasted_iota(jnp.int32, (_L, n), 1)
        return (idx_fn(row) == col).astype(jnp.float32)

    pos = (
        lax.dot(onehot(lambda r: r // _HW, _T), tbl[0, :_T],
                preferred_element_type=jnp.float32)
        + lax.dot(onehot(lambda r: (r // _W) % _H, _H), tbl[1, :_H],
                  preferred_element_type=jnp.float32)
        + lax.dot(onehot(lambda r: r % _W, _W), tbl[2, :_W],
                  preferred_element_type=jnp.float32)
    )
    mean = jnp.mean(pos, axis=-1, keepdims=True)
    c = pos - mean
    var = jnp.mean(c * c, axis=-1, keepdims=True)
    pos_ref[:] = c * lax.rsqrt(var + _EPS) * tbl[3, :1] + tbl[4, :1]

    for i in range(_B):
        s = i % _NBUF
        in_h[i].wait()
        buf[s] = buf[s] + pos_ref[:]
        out_h[i] = pltpu.make_async_copy(buf.at[s], o_ref.at[i],
                                         out_sem.at[s])
        out_h[i].start()
        n = i + _PREF + 1
        if n < _B:
            sn = n % _NBUF
            if n >= _NBUF:
                out_h[n - _NBUF].wait()
            in_h[n] = pltpu.make_async_copy(x_ref.at[n], buf.at[sn],
                                            in_sem.at[sn])
            in_h[n].start()
    for i in range(_B - _NBUF, _B):
        out_h[i].wait()


def kernel(inputs, temporal_emb, vertical_emb, horizontal_emb, gamma, beta,
           dimensions):
    out = pl.pallas_call(
        _stream_kernel,
        in_specs=[pl.BlockSpec(memory_space=pltpu.MemorySpace.HBM)] * 6,
        out_specs=pl.BlockSpec(memory_space=pltpu.MemorySpace.HBM),
        out_shape=jax.ShapeDtypeStruct((_B, _L, _D), jnp.float32),
        scratch_shapes=[
            pltpu.VMEM((_NBUF, _L, _D), jnp.float32),
            pltpu.VMEM((_L, _D), jnp.float32),
            pltpu.VMEM((5, 14, _D), jnp.float32),
            pltpu.SemaphoreType.DMA((_NBUF,)),
            pltpu.SemaphoreType.DMA((_NBUF,)),
            pltpu.SemaphoreType.DMA,
        ],
        compiler_params=pltpu.CompilerParams(
            vmem_limit_bytes=64 * 1024 * 1024,
        ),
    )(inputs, temporal_emb, vertical_emb, horizontal_emb,
      gamma.reshape(1, _D), beta.reshape(1, _D))
    return out


# speedup vs baseline: 1.0272x; 1.0272x over previous
import jax
import jax.numpy as jnp
from jax import lax
from jax.experimental import pallas as pl
from jax.experimental.pallas import tpu as pltpu

_B, _T, _H, _W, _D = 8, 8, 14, 14, 768
_HW = _H * _W
_L = _T * _HW
_EPS = 1e-06
_NBUF = 6
_PREF = 3


def _stream_kernel(x_ref, te_ref, ve_ref, he_ref, g_ref, b_ref, o_ref,
                   buf, pos_ref, tbl, in_sem, out_sem, tbl_sem):
    in_h = [None] * _B
    out_h = [None] * _B
    for i in range(_PREF + 1):
        in_h[i] = pltpu.make_async_copy(x_ref.at[i], buf.at[i], in_sem.at[i])
        in_h[i].start()

    cps = [
        pltpu.make_async_copy(te_ref, tbl.at[0, pl.ds(0, _T)], tbl_sem),
        pltpu.make_async_copy(ve_ref, tbl.at[1, pl.ds(0, _H)], tbl_sem),
        pltpu.make_async_copy(he_ref, tbl.at[2, pl.ds(0, _W)], tbl_sem),
        pltpu.make_async_copy(g_ref, tbl.at[3, pl.ds(0, 1)], tbl_sem),
        pltpu.make_async_copy(b_ref, tbl.at[4, pl.ds(0, 1)], tbl_sem),
    ]
    for cp in cps:
        cp.start()
    for cp in cps:
        cp.wait()

    def onehot(idx_fn, n):
        row = lax.broadcasted_iota(jnp.int32, (_L, n), 0)
        col = lax.broadcasted_iota(jnp.int32, (_L, n), 1)
        return (idx_fn(row) == col).astype(jnp.float32)

    pos = (
        lax.dot(onehot(lambda r: r // _HW, _T), tbl[0, :_T],
                preferred_element_type=jnp.float32)
        + lax.dot(onehot(lambda r: (r // _W) % _H, _H), tbl[1, :_H],
                  preferred_element_type=jnp.float32)
        + lax.dot(onehot(lambda r: r % _W, _W), tbl[2, :_W],
                  preferred_element_type=jnp.float32)
    )
    mean = jnp.mean(pos, axis=-1, keepdims=True)
    c = pos - mean
    var = jnp.mean(c * c, axis=-1, keepdims=True)
    pos_ref[:] = c * lax.rsqrt(var + _EPS) * tbl[3, :1] + tbl[4, :1]

    for i in range(_B):
        s = i % _NBUF
        in_h[i].wait()
        buf[s] = buf[s] + pos_ref[:]
        out_h[i] = pltpu.make_async_copy(buf.at[s], o_ref.at[i],
                                         out_sem.at[s])
        out_h[i].start()
        n = i + _PREF + 1
        if n < _B:
            sn = n % _NBUF
            if n >= _NBUF:
                out_h[n - _NBUF].wait()
            in_h[n] = pltpu.make_async_copy(x_ref.at[n], buf.at[sn],
                                            in_sem.at[sn])
            in_h[n].start()
    for i in range(_B - _NBUF, _B):
        out_h[i].wait()


def kernel(inputs, temporal_emb, vertical_emb, horizontal_emb, gamma, beta,
           dimensions):
    out = pl.pallas_call(
        _stream_kernel,
        in_specs=[pl.BlockSpec(memory_space=pltpu.MemorySpace.HBM)] * 6,
        out_specs=pl.BlockSpec(memory_space=pltpu.MemorySpace.HBM),
        out_shape=jax.ShapeDtypeStruct((_B, _L, _D), jnp.float32),
        scratch_shapes=[
            pltpu.VMEM((_NBUF, _L, _D), jnp.float32),
            pltpu.VMEM((_L, _D), jnp.float32),
            pltpu.VMEM((5, 14, _D), jnp.float32),
            pltpu.SemaphoreType.DMA((_NBUF,)),
            pltpu.SemaphoreType.DMA((_NBUF,)),
            pltpu.SemaphoreType.DMA,
        ],
        compiler_params=pltpu.CompilerParams(
            vmem_limit_bytes=64 * 1024 * 1024,
        ),
    )(inputs, temporal_emb, vertical_emb, horizontal_emb,
      gamma.reshape(1, _D), beta.reshape(1, _D))
    return out


# speedup vs baseline: 1.0635x; 1.0354x over previous
import jax
import jax.numpy as jnp
from jax.experimental import pallas as pl
from jax.experimental.pallas import tpu as pltpu

_B, _T, _H, _W, _D = 8, 8, 14, 14, 768
_HW = _H * _W
_L = _T * _HW
_EPS = 1e-06
_BL = 1568
_NJ = _L // _BL
_BB = 2


def _fused_kernel(x_ref, te_ref, ve_ref, he_ref, g_ref, b_ref, o_ref,
                  pos_ref):
    b = pl.program_id(0)
    j = pl.program_id(1)

    @pl.when((b == 0) & (j == 0))
    def _build_pos():
        def onehot(idx_fn, n):
            row = jax.lax.broadcasted_iota(jnp.int32, (_L, n), 0)
            col = jax.lax.broadcasted_iota(jnp.int32, (_L, n), 1)
            return (idx_fn(row) == col).astype(jnp.float32)

        pos = (
            jax.lax.dot(onehot(lambda r: r // _HW, _T), te_ref[:],
                        preferred_element_type=jnp.float32)
            + jax.lax.dot(onehot(lambda r: (r // _W) % _H, _H), ve_ref[:],
                          preferred_element_type=jnp.float32)
            + jax.lax.dot(onehot(lambda r: r % _W, _W), he_ref[:],
                          preferred_element_type=jnp.float32)
        )
        mean = jnp.mean(pos, axis=-1, keepdims=True)
        c = pos - mean
        var = jnp.mean(c * c, axis=-1, keepdims=True)
        pos_ref[:] = c * jax.lax.rsqrt(var + _EPS) * g_ref[:] + b_ref[:]

    o_ref[:] = x_ref[:] + pos_ref[pl.ds(j * _BL, _BL), :][None]


def kernel(inputs, temporal_emb, vertical_emb, horizontal_emb, gamma, beta,
           dimensions):
    g = gamma.reshape(1, _D)
    be = beta.reshape(1, _D)
    out = pl.pallas_call(
        _fused_kernel,
        grid=(_B // _BB, _NJ),
        in_specs=[
            pl.BlockSpec((_BB, _BL, _D), lambda b, j: (b, j, 0)),
            pl.BlockSpec((_T, _D), lambda b, j: (0, 0)),
            pl.BlockSpec((_H, _D), lambda b, j: (0, 0)),
            pl.BlockSpec((_W, _D), lambda b, j: (0, 0)),
            pl.BlockSpec((1, _D), lambda b, j: (0, 0)),
            pl.BlockSpec((1, _D), lambda b, j: (0, 0)),
        ],
        out_specs=pl.BlockSpec((_BB, _BL, _D), lambda b, j: (b, j, 0)),
        out_shape=jax.ShapeDtypeStruct((_B, _L, _D), jnp.float32),
        scratch_shapes=[
            pltpu.VMEM((_L, _D), jnp.float32),
        ],
        compiler_params=pltpu.CompilerParams(
            dimension_semantics=("arbitrary", "arbitrary"),
        ),
    )(inputs, temporal_emb, vertical_emb, horizontal_emb, g, be)
    return out
